# baseline (speedup 1.0000x reference)
import jax
import jax.numpy as jnp
from jax import lax
from jax.experimental import pallas as pl
from jax.experimental.pallas import tpu as pltpu

_NNZ = 4194304
_KEY_LO = 42
_LANES = 1024
_VROWS = _NNZ // _LANES
_IROWS = 2 * _NNZ // _LANES
_GRID = 16
_VB = _VROWS // _GRID
_IB = _IROWS // _GRID


def _rotl(x, d):
    return lax.shift_left(x, jnp.uint32(d)) | lax.shift_right_logical(
        x, jnp.uint32(32 - d))


def _threefry_bits(cnt):
    ks0 = jnp.uint32(0)
    ks1 = jnp.uint32(_KEY_LO)
    ks2 = jnp.uint32(0x1BD11BDA) ^ ks0 ^ ks1
    rot_a = (13, 15, 26, 6)
    rot_b = (17, 29, 16, 24)

    def four_rounds(x0, x1, rots):
        for r in rots:
            x0 = x0 + x1
            x1 = _rotl(x1, r) ^ x0
        return x0, x1

    x0 = jnp.zeros_like(cnt) + ks0
    x1 = cnt + ks1
    x0, x1 = four_rounds(x0, x1, rot_a)
    x0 += ks1
    x1 += ks2 + jnp.uint32(1)
    x0, x1 = four_rounds(x0, x1, rot_b)
    x0 += ks2
    x1 += ks0 + jnp.uint32(2)
    x0, x1 = four_rounds(x0, x1, rot_a)
    x0 += ks0
    x1 += ks1 + jnp.uint32(3)
    x0, x1 = four_rounds(x0, x1, rot_b)
    x0 += ks1
    x1 += ks2 + jnp.uint32(4)
    x0, x1 = four_rounds(x0, x1, rot_a)
    x0 += ks2
    x1 += ks0 + jnp.uint32(5)
    return x0 ^ x1


def _body(v_ref, i_ref, ov_ref, oi_ref):
    oi_ref[...] = i_ref[...]

    b = pl.program_id(0)
    base = (b * (_VB * _LANES)).astype(jnp.uint32)
    r = lax.broadcasted_iota(jnp.uint32, (_VB, _LANES), 0)
    c = lax.broadcasted_iota(jnp.uint32, (_VB, _LANES), 1)
    cnt = base + r * jnp.uint32(_LANES) + c
    bits = _threefry_bits(cnt)
    keep = bits < jnp.uint32(0x80000000)
    v = v_ref[...]
    ov_ref[...] = jnp.clip(jnp.where(keep, v * 2.0, 0.0), -1000000.0, 1000000.0)


def kernel(indices, values):
    v2d = values.reshape(_VROWS, _LANES)
    i2d = indices.reshape(_IROWS, _LANES)
    out_v, out_i = pl.pallas_call(
        _body,
        grid=(_GRID,),
        in_specs=[
            pl.BlockSpec((_VB, _LANES), lambda b: (b, 0)),
            pl.BlockSpec((_IB, _LANES), lambda b: (b, 0)),
        ],
        out_specs=[
            pl.BlockSpec((_VB, _LANES), lambda b: (b, 0)),
            pl.BlockSpec((_IB, _LANES), lambda b: (b, 0)),
        ],
        out_shape=[
            jax.ShapeDtypeStruct((_VROWS, _LANES), jnp.float32),
            jax.ShapeDtypeStruct((_IROWS, _LANES), jnp.int32),
        ],
        compiler_params=pltpu.CompilerParams(
            dimension_semantics=("arbitrary",)),
    )(v2d, i2d)
    return out_i.reshape(2, _NNZ), out_v.reshape(_NNZ)


# speedup vs baseline: 3.4968x; 3.4968x over previous
import jax
import jax.numpy as jnp
from jax import lax
from jax.experimental import pallas as pl
from jax.experimental.pallas import tpu as pltpu

_NNZ = 4194304
_KEY_LO = 42
_GRID = 16
_VB = _NNZ // _GRID
_IB = _NNZ // _GRID


def _rotl(x, d):
    return lax.shift_left(x, jnp.uint32(d)) | lax.shift_right_logical(
        x, jnp.uint32(32 - d))


def _threefry_bits(cnt):
    ks0 = jnp.uint32(0)
    ks1 = jnp.uint32(_KEY_LO)
    ks2 = jnp.uint32(0x1BD11BDA) ^ ks0 ^ ks1
    rot_a = (13, 15, 26, 6)
    rot_b = (17, 29, 16, 24)

    def four_rounds(x0, x1, rots):
        for r in rots:
            x0 = x0 + x1
            x1 = _rotl(x1, r) ^ x0
        return x0, x1

    x0 = jnp.zeros_like(cnt) + ks0
    x1 = cnt + ks1
    x0, x1 = four_rounds(x0, x1, rot_a)
    x0 += ks1
    x1 += ks2 + jnp.uint32(1)
    x0, x1 = four_rounds(x0, x1, rot_b)
    x0 += ks2
    x1 += ks0 + jnp.uint32(2)
    x0, x1 = four_rounds(x0, x1, rot_a)
    x0 += ks0
    x1 += ks1 + jnp.uint32(3)
    x0, x1 = four_rounds(x0, x1, rot_b)
    x0 += ks1
    x1 += ks2 + jnp.uint32(4)
    x0, x1 = four_rounds(x0, x1, rot_a)
    x0 += ks2
    x1 += ks0 + jnp.uint32(5)
    return x0 ^ x1


_LANES = 128
_VROWS = _NNZ // _LANES
_VBR = _VROWS // _GRID


def _body(v_ref, i_ref, ov_ref, oi_ref):
    oi_ref[...] = i_ref[...]

    b = pl.program_id(0)
    base = (b * _VB).astype(jnp.uint32)
    r = lax.broadcasted_iota(jnp.uint32, (_VBR, _LANES), 0)
    c = lax.broadcasted_iota(jnp.uint32, (_VBR, _LANES), 1)
    cnt = base + (lax.shift_left(r, jnp.uint32(7)) | c)
    bits = _threefry_bits(cnt)
    keep = bits < jnp.uint32(0x80000000)
    v = v_ref[...]
    ov_ref[...] = jnp.clip(jnp.where(keep, v * 2.0, 0.0), -1000000.0, 1000000.0)


def kernel(indices, values):
    v2d = values.reshape(_VROWS, _LANES)
    out_v, out_i = pl.pallas_call(
        _body,
        grid=(_GRID,),
        in_specs=[
            pl.BlockSpec((_VBR, _LANES), lambda b: (b, 0)),
            pl.BlockSpec((2, _IB), lambda b: (0, b)),
        ],
        out_specs=[
            pl.BlockSpec((_VBR, _LANES), lambda b: (b, 0)),
            pl.BlockSpec((2, _IB), lambda b: (0, b)),
        ],
        out_shape=[
            jax.ShapeDtypeStruct((_VROWS, _LANES), jnp.float32),
            jax.ShapeDtypeStruct((2, _NNZ), jnp.int32),
        ],
        compiler_params=pltpu.CompilerParams(
            dimension_semantics=("arbitrary",)),
    )(v2d, indices)
    return out_i, out_v.reshape(_NNZ)
